# baseline (speedup 1.0000x reference)
import functools

import jax
import jax.numpy as jnp
from jax import lax
from jax.experimental import pallas as pl
from jax.experimental.pallas import tpu as pltpu
from jax.experimental.pallas import tpu_sc as plsc

N = 10000
E = 320000
HID = 100
W = 128
DEG_COL = 100

NC, NS = 2, 16
NW = NC * NS
B_E = 96
N_CH = 106
EPT = B_E * N_CH
E_PAD = EPT * NW
SHIFT = 14
MASK = (1 << SHIFT) - 1
N_PAD = 10112
ROWS_PT = N_PAD // NS

_mesh = plsc.VectorSubcoreMesh(core_axis_name="c", subcore_axis_name="s")


@functools.partial(
    pl.kernel,
    mesh=_mesh,
    out_type=jax.ShapeDtypeStruct((NC, N_PAD, W), jnp.float32),
    scratch_types=[
        pltpu.VMEM((N_CH, B_E), jnp.int32),
        pltpu.VMEM((2, B_E), jnp.int32),
        pltpu.VMEM((2, B_E), jnp.int32),
        pltpu.VMEM((2, B_E, W), jnp.float32),
        pltpu.VMEM_SHARED((N_PAD, W), jnp.float32),
        [pltpu.SemaphoreType.DMA] * 2,
    ],
)
def _sc_segment_sum(y_hbm, sd_hbm, zeros_hbm, out_hbm,
                    sd_v, src_v, dst_v, rows_v, acc_sh, sems):
    c = lax.axis_index("c")
    s = lax.axis_index("s")
    wid = s * NC + c

    pltpu.sync_copy(zeros_hbm.at[pl.ds(s * ROWS_PT, ROWS_PT)],
                    acc_sh.at[pl.ds(s * ROWS_PT, ROWS_PT)])
    pltpu.sync_copy(sd_hbm.at[wid], sd_v)
    plsc.subcore_barrier()

    def unpack(j, b):
        for k in range(B_E // 16):
            p = sd_v[j, pl.ds(k * 16, 16)]
            src_v[b, pl.ds(k * 16, 16)] = p >> SHIFT
            dst_v[b, pl.ds(k * 16, 16)] = p & MASK

    def start_gather(b):
        pltpu.async_copy(y_hbm.at[src_v.at[b]], rows_v.at[b], sems[b])

    def wait_buf(b):
        pltpu.make_async_copy(y_hbm.at[src_v.at[0]], rows_v.at[b],
                              sems[b]).wait()

    def scatter(b):
        pltpu.sync_copy(rows_v.at[b], acc_sh.at[dst_v.at[b]], add=True)

    unpack(0, 0)
    start_gather(0)
    unpack(1, 1)
    start_gather(1)

    def body(i, carry):
        j0 = 2 * i
        wait_buf(0)
        scatter(0)
        unpack(j0 + 2, 0)
        start_gather(0)
        wait_buf(1)
        scatter(1)
        unpack(j0 + 3, 1)
        start_gather(1)
        return carry

    lax.fori_loop(0, N_CH // 2 - 1, body, 0)
    wait_buf(0)
    scatter(0)
    wait_buf(1)
    scatter(1)

    plsc.subcore_barrier()
    pltpu.sync_copy(acc_sh.at[pl.ds(s * ROWS_PT, ROWS_PT)],
                    out_hbm.at[c, pl.ds(s * ROWS_PT, ROWS_PT)])


def _tc_pre1(x_ref, w1l_ref, w1r_ref, b1l_ref, y_ref, r_ref):
    x = x_ref[...]
    y = jnp.dot(x, w1l_ref[...], preferred_element_type=jnp.float32)
    ones = jnp.ones((x.shape[0], 1), jnp.float32)
    zpad = jnp.zeros((x.shape[0], W - HID - 1), jnp.float32)
    y_ref[...] = jnp.concatenate([y, ones, zpad], axis=1)
    r_ref[...] = (jnp.dot(x, w1r_ref[...], preferred_element_type=jnp.float32)
                  + b1l_ref[...])


def _tc_mid(p_ref, r1_ref, w2l_ref, w2r_ref, b2l_ref,
            y2_ref, r2_ref, deg_ref):
    p = p_ref[...]
    acc = p[0] + p[1]
    deg = jnp.maximum(acc[:, DEG_COL:DEG_COL + 1], 1.0)
    h = jnp.maximum(acc[:, :HID] / deg + r1_ref[...], 0.0)
    y2 = jnp.dot(h, w2l_ref[...], preferred_element_type=jnp.float32)
    zpad = jnp.zeros((h.shape[0], W - HID), jnp.float32)
    y2_ref[...] = jnp.concatenate([y2, zpad], axis=1)
    r2_ref[...] = (jnp.dot(h, w2r_ref[...], preferred_element_type=jnp.float32)
                   + b2l_ref[...])
    deg_ref[...] = deg


def _tc_post(p_ref, r2_ref, deg_ref, o_ref):
    p = p_ref[...]
    acc = p[0] + p[1]
    o_ref[...] = acc[:, :HID] / deg_ref[...] + r2_ref[...]


def kernel(x, ei, W1_l, b1_l, W1_r, W2_l, b2_l, W2_r):
    src = ei[0].astype(jnp.int32)
    dst = ei[1].astype(jnp.int32)
    pad = E_PAD - E
    sd = (src << SHIFT) | dst
    sd_p = jnp.concatenate(
        [sd, jnp.full((pad,), N, jnp.int32)]).reshape(NW, N_CH, B_E)
    zeros = jnp.zeros((N_PAD, W), jnp.float32)

    y1, r1 = pl.pallas_call(
        _tc_pre1,
        out_shape=[
            jax.ShapeDtypeStruct((N, W), jnp.float32),
            jax.ShapeDtypeStruct((N, HID), jnp.float32),
        ],
    )(x, W1_l, W1_r, b1_l.reshape(1, HID))

    p1 = _sc_segment_sum(y1, sd_p, zeros)

    y2, r2, deg = pl.pallas_call(
        _tc_mid,
        out_shape=[
            jax.ShapeDtypeStruct((N, W), jnp.float32),
            jax.ShapeDtypeStruct((N, HID), jnp.float32),
            jax.ShapeDtypeStruct((N, 1), jnp.float32),
        ],
    )(p1[:, :N, :], r1, W2_l, W2_r, b2_l.reshape(1, HID))

    p2 = _sc_segment_sum(y2, sd_p, zeros)

    out = pl.pallas_call(
        _tc_post,
        out_shape=jax.ShapeDtypeStruct((N, HID), jnp.float32),
    )(p2[:, :N, :], r2, deg)
    return out


# speedup vs baseline: 1.0030x; 1.0030x over previous
import functools

import jax
import jax.numpy as jnp
from jax import lax
from jax.experimental import pallas as pl
from jax.experimental.pallas import tpu as pltpu
from jax.experimental.pallas import tpu_sc as plsc

N = 10000
E = 320000
HID = 100
W = 128
DEG_COL = 100

NC, NS = 2, 16
NW = NC * NS
B_E = 96
N_CH = 106
EPT = B_E * N_CH
E_PAD = EPT * NW
SHIFT = 14
MASK = (1 << SHIFT) - 1
N_PAD = 10112
ROWS_PT = N_PAD // NS

_mesh = plsc.VectorSubcoreMesh(core_axis_name="c", subcore_axis_name="s")


@functools.partial(
    pl.kernel,
    mesh=_mesh,
    out_type=jax.ShapeDtypeStruct((NC, N_PAD, W), jnp.float32),
    scratch_types=[
        pltpu.VMEM((EPT,), jnp.int32),
        pltpu.VMEM((2, B_E), jnp.int32),
        pltpu.VMEM((2, B_E), jnp.int32),
        pltpu.VMEM((2, B_E, W), jnp.float32),
        pltpu.VMEM_SHARED((N_PAD, W), jnp.float32),
        [pltpu.SemaphoreType.DMA] * 2,
    ],
)
def _sc_segment_sum(y_hbm, sd_hbm, zeros_hbm, out_hbm,
                    sd_v, src_v, dst_v, rows_v, acc_sh, sems):
    c = lax.axis_index("c")
    s = lax.axis_index("s")
    wid = s * NC + c

    pltpu.sync_copy(zeros_hbm.at[pl.ds(s * ROWS_PT, ROWS_PT)],
                    acc_sh.at[pl.ds(s * ROWS_PT, ROWS_PT)])
    pltpu.sync_copy(sd_hbm.at[wid], sd_v)
    plsc.subcore_barrier()

    def unpack(j, b):
        for k in range(B_E // 16):
            p = sd_v[pl.ds(j * B_E + k * 16, 16)]
            src_v[b, pl.ds(k * 16, 16)] = p >> SHIFT
            dst_v[b, pl.ds(k * 16, 16)] = p & MASK

    def start_gather(b):
        pltpu.async_copy(y_hbm.at[src_v.at[b]], rows_v.at[b], sems[b])

    def wait_buf(b):
        pltpu.make_async_copy(y_hbm.at[src_v.at[0]], rows_v.at[b],
                              sems[b]).wait()

    def scatter(b):
        pltpu.sync_copy(rows_v.at[b], acc_sh.at[dst_v.at[b]], add=True)

    unpack(0, 0)
    start_gather(0)
    unpack(1, 1)
    start_gather(1)

    def body(i, carry):
        j0 = 2 * i
        wait_buf(0)
        scatter(0)
        unpack(j0 + 2, 0)
        start_gather(0)
        wait_buf(1)
        scatter(1)
        unpack(j0 + 3, 1)
        start_gather(1)
        return carry

    lax.fori_loop(0, N_CH // 2 - 1, body, 0)
    wait_buf(0)
    scatter(0)
    wait_buf(1)
    scatter(1)

    plsc.subcore_barrier()
    pltpu.sync_copy(acc_sh.at[pl.ds(s * ROWS_PT, ROWS_PT)],
                    out_hbm.at[c, pl.ds(s * ROWS_PT, ROWS_PT)])


def _tc_pre1(x_ref, w1l_ref, w1r_ref, b1l_ref, y_ref, r_ref):
    x = x_ref[...]
    y = jnp.dot(x, w1l_ref[...], preferred_element_type=jnp.float32)
    ones = jnp.ones((x.shape[0], 1), jnp.float32)
    zpad = jnp.zeros((x.shape[0], W - HID - 1), jnp.float32)
    y_ref[...] = jnp.concatenate([y, ones, zpad], axis=1)
    r_ref[...] = (jnp.dot(x, w1r_ref[...], preferred_element_type=jnp.float32)
                  + b1l_ref[...])


def _tc_mid(p_ref, r1_ref, w2l_ref, w2r_ref, b2l_ref,
            y2_ref, r2_ref, deg_ref):
    p = p_ref[...]
    acc = p[0] + p[1]
    deg = jnp.maximum(acc[:, DEG_COL:DEG_COL + 1], 1.0)
    h = jnp.maximum(acc[:, :HID] / deg + r1_ref[...], 0.0)
    y2 = jnp.dot(h, w2l_ref[...], preferred_element_type=jnp.float32)
    zpad = jnp.zeros((h.shape[0], W - HID), jnp.float32)
    y2_ref[...] = jnp.concatenate([y2, zpad], axis=1)
    r2_ref[...] = (jnp.dot(h, w2r_ref[...], preferred_element_type=jnp.float32)
                   + b2l_ref[...])
    deg_ref[...] = deg


def _tc_post(p_ref, r2_ref, deg_ref, o_ref):
    p = p_ref[...]
    acc = p[0] + p[1]
    o_ref[...] = acc[:, :HID] / deg_ref[...] + r2_ref[...]


def kernel(x, ei, W1_l, b1_l, W1_r, W2_l, b2_l, W2_r):
    src = ei[0].astype(jnp.int32)
    dst = ei[1].astype(jnp.int32)
    pad = E_PAD - E
    sd = (src << SHIFT) | dst
    sd_p = jnp.concatenate(
        [sd, jnp.full((pad,), N, jnp.int32)]).reshape(NW, EPT)
    zeros = jnp.zeros((N_PAD, W), jnp.float32)

    y1, r1 = pl.pallas_call(
        _tc_pre1,
        out_shape=[
            jax.ShapeDtypeStruct((N, W), jnp.float32),
            jax.ShapeDtypeStruct((N, HID), jnp.float32),
        ],
    )(x, W1_l, W1_r, b1_l.reshape(1, HID))

    p1 = _sc_segment_sum(y1, sd_p, zeros)

    y2, r2, deg = pl.pallas_call(
        _tc_mid,
        out_shape=[
            jax.ShapeDtypeStruct((N, W), jnp.float32),
            jax.ShapeDtypeStruct((N, HID), jnp.float32),
            jax.ShapeDtypeStruct((N, 1), jnp.float32),
        ],
    )(p1[:, :N, :], r1, W2_l, W2_r, b2_l.reshape(1, HID))

    p2 = _sc_segment_sum(y2, sd_p, zeros)

    out = pl.pallas_call(
        _tc_post,
        out_shape=jax.ShapeDtypeStruct((N, HID), jnp.float32),
    )(p2[:, :N, :], r2, deg)
    return out
